# baseline (speedup 1.0000x reference)
import jax
import jax.numpy as jnp
from jax.experimental import pallas as pl
from jax.experimental.pallas import tpu as pltpu

HNF = 128
NEF = 16
MSG = 128
PASSES = 3
OUT_F = 128
PACK = 128 // NEF


def _mpnn_kernel(nodes_ref, edges_ref, w_n_ref, w_bd_ref, r16_ref, q_ref,
                 b_msg_ref,
                 w_uh_ref, w_um_ref, b_upd_ref, wg_h_ref, wg_n_ref,
                 w_out_ref, out_ref):
    g, n, _ = nodes_ref.shape
    rows = n * n // PACK
    nodes_b = nodes_ref[...].reshape(g * n, HNF)
    ef = edges_ref[...].reshape(g * rows, PACK * NEF)

    a0 = jnp.dot(nodes_b, w_n_ref[...], preferred_element_type=jnp.float32)
    a0 = a0 + b_msg_ref[...]
    a = a0.reshape(g, PACK, PACK * MSG)

    ew2 = jnp.dot(ef, w_bd_ref[...], preferred_element_type=jnp.float32)
    ew = ew2.reshape(g, n, PACK, PACK * MSG)

    terms = jnp.tanh(ew + a[:, None, :, :])

    t2 = terms.reshape(g * rows, PACK * MSG)
    s = t2[:, 0:MSG]
    for k in range(1, PACK):
        s = s + t2[:, k * MSG:(k + 1) * MSG]
    diff_sum = jnp.sum(s.reshape(g * n, PACK, MSG), axis=1)

    adj_rep = jnp.dot(ef, r16_ref[...], preferred_element_type=jnp.float32)
    row_q = jax.lax.broadcasted_iota(jnp.int32, (g * rows, n), 0) % PACK
    lane_k = jax.lax.broadcasted_iota(jnp.int32, (g * rows, n), 1) // PACK
    vwide = jnp.where((row_q == lane_k) & (adj_rep != 0.0), 1.0, 0.0)
    tnat = jnp.tanh(a0)
    ones_b = jnp.ones((n, MSG), jnp.float32)
    corr = []
    deg = []
    for gi in range(g):
        vv = jnp.dot(q_ref[...], vwide[gi * rows:(gi + 1) * rows],
                     preferred_element_type=jnp.float32)
        rhs = jnp.concatenate(
            [tnat[gi * n:(gi + 1) * n], ones_b], axis=1)
        cd = jnp.dot(vv - 1.0, rhs,
                     precision=jax.lax.Precision.HIGHEST,
                     preferred_element_type=jnp.float32)
        corr.append(cd[:, :MSG])
        deg.append(cd[:, MSG:])
    messages = diff_sum + jnp.concatenate(corr, axis=0)
    node_mask = jnp.concatenate(deg, axis=0) > -float(n)

    hidden = nodes_b
    for _ in range(PASSES):
        upd = jnp.tanh(
            jnp.dot(hidden, w_uh_ref[...], preferred_element_type=jnp.float32)
            + jnp.dot(messages, w_um_ref[...], preferred_element_type=jnp.float32)
            + b_upd_ref[...])
        hidden = jnp.where(node_mask, upd, hidden)

    gate = jax.nn.sigmoid(
        jnp.dot(hidden, wg_h_ref[...], preferred_element_type=jnp.float32)
        + jnp.dot(nodes_b, wg_n_ref[...], preferred_element_type=jnp.float32))
    contrib = gate * jnp.dot(hidden, w_out_ref[...],
                             preferred_element_type=jnp.float32)
    contrib = jnp.where(node_mask, contrib, 0.0)
    out_ref[...] = jnp.sum(contrib.reshape(g, n, OUT_F), axis=1, keepdims=True)


G_BATCH = 8


def _build_call(B, N):
    G = G_BATCH
    ROWS = N * N // PACK
    return pl.pallas_call(
        _mpnn_kernel,
        grid=(B // G,),
        in_specs=[
            pl.BlockSpec((G, N, HNF), lambda b: (b, 0, 0)),
            pl.BlockSpec((G, ROWS, PACK * NEF), lambda b: (b, 0, 0)),
            pl.BlockSpec((HNF, MSG), lambda b: (0, 0)),
            pl.BlockSpec((PACK * NEF, PACK * MSG), lambda b: (0, 0)),
            pl.BlockSpec((PACK * NEF, N), lambda b: (0, 0)),
            pl.BlockSpec((N, N * N // PACK), lambda b: (0, 0)),
            pl.BlockSpec((1, MSG), lambda b: (0, 0)),
            pl.BlockSpec((HNF, HNF), lambda b: (0, 0)),
            pl.BlockSpec((MSG, HNF), lambda b: (0, 0)),
            pl.BlockSpec((1, HNF), lambda b: (0, 0)),
            pl.BlockSpec((HNF, OUT_F), lambda b: (0, 0)),
            pl.BlockSpec((HNF, OUT_F), lambda b: (0, 0)),
            pl.BlockSpec((HNF, OUT_F), lambda b: (0, 0)),
        ],
        out_specs=pl.BlockSpec((G, 1, OUT_F), lambda b: (b, 0, 0)),
        out_shape=jax.ShapeDtypeStruct((B, 1, OUT_F), jnp.float32),
        compiler_params=pltpu.CompilerParams(
            dimension_semantics=("parallel",),
        ),
    )


def kernel(nodes, edges, W_msg, b_msg, W_upd_h, W_upd_m, b_upd, W_gate, W_out):
    B, N, F = nodes.shape
    w_n = W_msg[:F]
    w_e = W_msg[F:]
    wg_h = W_gate[:F]
    wg_n = W_gate[F:]
    eye8 = jnp.eye(PACK, dtype=jnp.float32)
    w_bd = jnp.einsum("pq,ef->peqf", eye8, w_e).reshape(PACK * NEF, PACK * MSG)
    r16 = (jnp.arange(PACK * NEF)[:, None] // NEF
           == jnp.arange(N)[None, :] % PACK).astype(jnp.float32)
    q_mat = (jnp.arange(N)[:, None]
             == jnp.arange(N * N // PACK)[None, :] // PACK).astype(jnp.float32)
    e_packed = edges.reshape(B, N * N // PACK, PACK * NEF)
    out = _build_call(B, N)(
        nodes, e_packed, w_n, w_bd, r16, q_mat, b_msg.reshape(1, MSG),
        W_upd_h, W_upd_m, b_upd.reshape(1, HNF), wg_h, wg_n, W_out)
    return out.reshape(B, OUT_F)


# speedup vs baseline: 1.0505x; 1.0505x over previous
import jax
import jax.numpy as jnp
from jax.experimental import pallas as pl
from jax.experimental.pallas import tpu as pltpu

HNF = 128
NEF = 16
MSG = 128
PASSES = 3
OUT_F = 128
PACK = 128 // NEF


def _mpnn_kernel(nodes_ref, edges_ref, w_n_ref, w_bd_ref, r16_ref, q_ref,
                 b_msg_ref,
                 w_uh_ref, w_um_ref, b_upd_ref, wg_h_ref, wg_n_ref,
                 w_out_ref, out_ref):
    g, n, _ = nodes_ref.shape
    rows = n * n // PACK
    nodes_b = nodes_ref[...].reshape(g * n, HNF)
    ef = edges_ref[...].reshape(g * rows, PACK * NEF)

    a0 = jnp.dot(nodes_b, w_n_ref[...], preferred_element_type=jnp.float32)
    a0 = a0 + b_msg_ref[...]
    a = a0.reshape(g, PACK, PACK * MSG)

    ew2 = jnp.dot(ef, w_bd_ref[...], preferred_element_type=jnp.float32)
    ew = ew2.reshape(g, n, PACK, PACK * MSG)

    terms = jnp.tanh(ew + a[:, None, :, :])

    t2 = terms.reshape(g * rows, PACK * MSG)
    s = t2[:, 0:MSG]
    for k in range(1, PACK):
        s = s + t2[:, k * MSG:(k + 1) * MSG]
    diff_sum = jnp.sum(s.reshape(g * n, PACK, MSG), axis=1)

    adj_rep = jnp.dot(ef, r16_ref[...], preferred_element_type=jnp.float32)
    row_q = jax.lax.broadcasted_iota(jnp.int32, (g * rows, n), 0) % PACK
    lane_k = jax.lax.broadcasted_iota(jnp.int32, (g * rows, n), 1) // PACK
    vwide = jnp.where((row_q == lane_k) & (adj_rep != 0.0), 1.0, 0.0)
    vv = jnp.sum(vwide.reshape(g * n, PACK, n), axis=1) - 1.0
    tnat = jnp.tanh(a0)
    ones_b = jnp.ones((n, MSG), jnp.float32)
    corr = []
    deg = []
    for gi in range(g):
        rhs = jnp.concatenate(
            [tnat[gi * n:(gi + 1) * n], ones_b], axis=1)
        cd = jnp.dot(vv[gi * n:(gi + 1) * n], rhs,
                     precision=jax.lax.Precision.HIGHEST,
                     preferred_element_type=jnp.float32)
        corr.append(cd[:, :MSG])
        deg.append(cd[:, MSG:])
    messages = diff_sum + jnp.concatenate(corr, axis=0)
    node_mask = jnp.concatenate(deg, axis=0) > -float(n)

    hidden = nodes_b
    for _ in range(PASSES):
        upd = jnp.tanh(
            jnp.dot(hidden, w_uh_ref[...], preferred_element_type=jnp.float32)
            + jnp.dot(messages, w_um_ref[...], preferred_element_type=jnp.float32)
            + b_upd_ref[...])
        hidden = jnp.where(node_mask, upd, hidden)

    gate = jax.nn.sigmoid(
        jnp.dot(hidden, wg_h_ref[...], preferred_element_type=jnp.float32)
        + jnp.dot(nodes_b, wg_n_ref[...], preferred_element_type=jnp.float32))
    contrib = gate * jnp.dot(hidden, w_out_ref[...],
                             preferred_element_type=jnp.float32)
    contrib = jnp.where(node_mask, contrib, 0.0)
    out_ref[...] = jnp.sum(contrib.reshape(g, n, OUT_F), axis=1, keepdims=True)


G_BATCH = 8


def _build_call(B, N):
    G = G_BATCH
    ROWS = N * N // PACK
    return pl.pallas_call(
        _mpnn_kernel,
        grid=(B // G,),
        in_specs=[
            pl.BlockSpec((G, N, HNF), lambda b: (b, 0, 0)),
            pl.BlockSpec((G, ROWS, PACK * NEF), lambda b: (b, 0, 0)),
            pl.BlockSpec((HNF, MSG), lambda b: (0, 0)),
            pl.BlockSpec((PACK * NEF, PACK * MSG), lambda b: (0, 0)),
            pl.BlockSpec((PACK * NEF, N), lambda b: (0, 0)),
            pl.BlockSpec((N, N * N // PACK), lambda b: (0, 0)),
            pl.BlockSpec((1, MSG), lambda b: (0, 0)),
            pl.BlockSpec((HNF, HNF), lambda b: (0, 0)),
            pl.BlockSpec((MSG, HNF), lambda b: (0, 0)),
            pl.BlockSpec((1, HNF), lambda b: (0, 0)),
            pl.BlockSpec((HNF, OUT_F), lambda b: (0, 0)),
            pl.BlockSpec((HNF, OUT_F), lambda b: (0, 0)),
            pl.BlockSpec((HNF, OUT_F), lambda b: (0, 0)),
        ],
        out_specs=pl.BlockSpec((G, 1, OUT_F), lambda b: (b, 0, 0)),
        out_shape=jax.ShapeDtypeStruct((B, 1, OUT_F), jnp.float32),
        compiler_params=pltpu.CompilerParams(
            dimension_semantics=("parallel",),
        ),
    )


def kernel(nodes, edges, W_msg, b_msg, W_upd_h, W_upd_m, b_upd, W_gate, W_out):
    B, N, F = nodes.shape
    w_n = W_msg[:F]
    w_e = W_msg[F:]
    wg_h = W_gate[:F]
    wg_n = W_gate[F:]
    eye8 = jnp.eye(PACK, dtype=jnp.float32)
    w_bd = jnp.einsum("pq,ef->peqf", eye8, w_e).reshape(PACK * NEF, PACK * MSG)
    r16 = (jnp.arange(PACK * NEF)[:, None] // NEF
           == jnp.arange(N)[None, :] % PACK).astype(jnp.float32)
    q_mat = (jnp.arange(N)[:, None]
             == jnp.arange(N * N // PACK)[None, :] // PACK).astype(jnp.float32)
    e_packed = edges.reshape(B, N * N // PACK, PACK * NEF)
    out = _build_call(B, N)(
        nodes, e_packed, w_n, w_bd, r16, q_mat, b_msg.reshape(1, MSG),
        W_upd_h, W_upd_m, b_upd.reshape(1, HNF), wg_h, wg_n, W_out)
    return out.reshape(B, OUT_F)


# speedup vs baseline: 1.0591x; 1.0083x over previous
import jax
import jax.numpy as jnp
from jax.experimental import pallas as pl
from jax.experimental.pallas import tpu as pltpu

HNF = 128
NEF = 16
MSG = 128
PASSES = 3
OUT_F = 128
PACK = 128 // NEF


def _mpnn_kernel(nodes_ref, edges_ref, w_n_ref, w_bd_ref, r16_ref,
                 b_msg_ref,
                 w_uh_ref, w_um_ref, b_upd_ref, wg_h_ref, wg_n_ref,
                 w_out_ref, out_ref):
    g, n, _ = nodes_ref.shape
    rows = n * n // PACK
    nodes_b = nodes_ref[...].reshape(g * n, HNF)
    ef = edges_ref[...].reshape(g * rows, PACK * NEF)

    a0 = jnp.dot(nodes_b, w_n_ref[...], preferred_element_type=jnp.float32)
    a0 = a0 + b_msg_ref[...]
    a = a0.reshape(g, PACK, PACK * MSG)

    ew2 = jnp.dot(ef.astype(jnp.bfloat16), w_bd_ref[...],
                  preferred_element_type=jnp.float32)
    ew = ew2.reshape(g, n, PACK, PACK * MSG)

    terms = jnp.tanh(ew + a[:, None, :, :])

    t2 = terms.reshape(g * rows, PACK * MSG)
    s = t2[:, 0:MSG]
    for k in range(1, PACK):
        s = s + t2[:, k * MSG:(k + 1) * MSG]
    diff_sum = jnp.sum(s.reshape(g * n, PACK, MSG), axis=1)

    adj_rep = jnp.dot(ef, r16_ref[...], preferred_element_type=jnp.float32)
    row_q = jax.lax.broadcasted_iota(jnp.int32, (g * rows, n), 0) % PACK
    lane_k = jax.lax.broadcasted_iota(jnp.int32, (g * rows, n), 1) // PACK
    vwide = jnp.where((row_q == lane_k) & (adj_rep != 0.0), 1.0, 0.0)
    vv = jnp.sum(vwide.reshape(g * n, PACK, n), axis=1) - 1.0
    tnat = jnp.tanh(a0)
    ones_b = jnp.ones((n, MSG), jnp.float32)
    corr = []
    deg = []
    for gi in range(g):
        rhs = jnp.concatenate(
            [tnat[gi * n:(gi + 1) * n], ones_b], axis=1)
        cd = jnp.dot(vv[gi * n:(gi + 1) * n], rhs,
                     precision=jax.lax.Precision.HIGHEST,
                     preferred_element_type=jnp.float32)
        corr.append(cd[:, :MSG])
        deg.append(cd[:, MSG:])
    messages = diff_sum + jnp.concatenate(corr, axis=0)
    node_mask = jnp.concatenate(deg, axis=0) > -float(n)

    hidden = nodes_b
    for _ in range(PASSES):
        upd = jnp.tanh(
            jnp.dot(hidden, w_uh_ref[...], preferred_element_type=jnp.float32)
            + jnp.dot(messages, w_um_ref[...], preferred_element_type=jnp.float32)
            + b_upd_ref[...])
        hidden = jnp.where(node_mask, upd, hidden)

    gate = jax.nn.sigmoid(
        jnp.dot(hidden, wg_h_ref[...], preferred_element_type=jnp.float32)
        + jnp.dot(nodes_b, wg_n_ref[...], preferred_element_type=jnp.float32))
    contrib = gate * jnp.dot(hidden, w_out_ref[...],
                             preferred_element_type=jnp.float32)
    contrib = jnp.where(node_mask, contrib, 0.0)
    out_ref[...] = jnp.sum(contrib.reshape(g, n, OUT_F), axis=1, keepdims=True)


G_BATCH = 8


def _build_call(B, N):
    G = G_BATCH
    ROWS = N * N // PACK
    return pl.pallas_call(
        _mpnn_kernel,
        grid=(B // G,),
        in_specs=[
            pl.BlockSpec((G, N, HNF), lambda b: (b, 0, 0)),
            pl.BlockSpec((G, ROWS, PACK * NEF), lambda b: (b, 0, 0)),
            pl.BlockSpec((HNF, MSG), lambda b: (0, 0)),
            pl.BlockSpec((PACK * NEF, PACK * MSG), lambda b: (0, 0)),
            pl.BlockSpec((PACK * NEF, N), lambda b: (0, 0)),
            pl.BlockSpec((1, MSG), lambda b: (0, 0)),
            pl.BlockSpec((HNF, HNF), lambda b: (0, 0)),
            pl.BlockSpec((MSG, HNF), lambda b: (0, 0)),
            pl.BlockSpec((1, HNF), lambda b: (0, 0)),
            pl.BlockSpec((HNF, OUT_F), lambda b: (0, 0)),
            pl.BlockSpec((HNF, OUT_F), lambda b: (0, 0)),
            pl.BlockSpec((HNF, OUT_F), lambda b: (0, 0)),
        ],
        out_specs=pl.BlockSpec((G, 1, OUT_F), lambda b: (b, 0, 0)),
        out_shape=jax.ShapeDtypeStruct((B, 1, OUT_F), jnp.float32),
        compiler_params=pltpu.CompilerParams(
            dimension_semantics=("parallel",),
        ),
    )


def kernel(nodes, edges, W_msg, b_msg, W_upd_h, W_upd_m, b_upd, W_gate, W_out):
    B, N, F = nodes.shape
    w_n = W_msg[:F]
    w_e = W_msg[F:]
    wg_h = W_gate[:F]
    wg_n = W_gate[F:]
    eye8 = jnp.eye(PACK, dtype=jnp.float32)
    w_bd = jnp.einsum("pq,ef->peqf", eye8, w_e).reshape(
        PACK * NEF, PACK * MSG).astype(jnp.bfloat16)
    r16 = (jnp.arange(PACK * NEF)[:, None] // NEF
           == jnp.arange(N)[None, :] % PACK).astype(jnp.float32)
    e_packed = edges.reshape(B, N * N // PACK, PACK * NEF)
    out = _build_call(B, N)(
        nodes, e_packed, w_n, w_bd, r16, b_msg.reshape(1, MSG),
        W_upd_h, W_upd_m, b_upd.reshape(1, HNF), wg_h, wg_n, W_out)
    return out.reshape(B, OUT_F)
